# baseline (speedup 1.0000x reference)
import functools
import math

import jax
import jax.numpy as jnp
from jax import lax
from jax.experimental import pallas as pl
from jax.experimental.pallas import tpu as pltpu
from jax.experimental.pallas import tpu_sc as plsc

NUM_EMBEDDINGS = 1000000
EMBED_DIM = 64
TOTAL_ROWS = 4096 * 200
NUM_WORKERS = 32
ROWS_PER_WORKER = TOTAL_ROWS // NUM_WORKERS
TILE_ROWS = 512
NUM_TILES = ROWS_PER_WORKER // TILE_ROWS
GATHER_CHUNK = 128
GATHERS_PER_TILE = TILE_ROWS // GATHER_CHUNK
IDX_COLS = 128
SCALE = math.sqrt(EMBED_DIM)
LANES = 16
VECS_PER_ROW = EMBED_DIM // LANES


def _perm(v, idx):
    return lax.gather(
        v,
        idx[:, None],
        dimension_numbers=lax.GatherDimensionNumbers(
            offset_dims=(), collapsed_slice_dims=(0,), start_index_map=(0,)),
        slice_sizes=(1,),
        mode=lax.GatherScatterMode.PROMISE_IN_BOUNDS,
    )


def _hsum(v):
    lanes = lax.iota(jnp.int32, LANES)
    for k in (8, 4, 2, 1):
        v = v + _perm(v, lanes ^ k)
    return v


def _rsqrt_newton(s):
    i = lax.bitcast_convert_type(s, jnp.int32)
    i = jnp.int32(0x5F3759DF) - lax.shift_right_arithmetic(i, 1)
    y = lax.bitcast_convert_type(i, jnp.float32)
    half = jnp.float32(0.5) * s
    for _ in range(3):
        y = y * (jnp.float32(1.5) - half * y * y)
    return y


def _make_kernel():
    mesh = plsc.VectorSubcoreMesh(core_axis_name="c", subcore_axis_name="s")

    @functools.partial(
        pl.kernel,
        mesh=mesh,
        compiler_params=pltpu.CompilerParams(use_tc_tiling_on_sc=False),
        out_type=jax.ShapeDtypeStruct((TOTAL_ROWS, EMBED_DIM), jnp.float32),
        scratch_types=[
            pltpu.VMEM((GATHERS_PER_TILE, GATHER_CHUNK), jnp.int32),
            pltpu.VMEM((TILE_ROWS, EMBED_DIM), jnp.float32),
            pltpu.SemaphoreType.DMA,
        ],
    )
    def emb_kernel(idx_hbm, table_hbm, out_hbm, idx_v, rows_v, sem):
        wid = lax.axis_index("s") * 2 + lax.axis_index("c")
        idx_row0 = wid * (ROWS_PER_WORKER // IDX_COLS)
        out_row0 = wid * ROWS_PER_WORKER

        def tile_body(t, _):
            pltpu.sync_copy(
                idx_hbm.at[pl.ds(idx_row0 + t * GATHERS_PER_TILE,
                                 GATHERS_PER_TILE)],
                idx_v,
            )
            copies = []
            for g in range(GATHERS_PER_TILE):
                copies.append(
                    pltpu.async_copy(
                        table_hbm.at[idx_v.at[g]],
                        rows_v.at[pl.ds(g * GATHER_CHUNK, GATHER_CHUNK)],
                        sem,
                    )
                )
            for c in copies:
                c.wait()

            def row_body(r, _):
                vs = [rows_v[r, pl.ds(k * LANES, LANES)]
                      for k in range(VECS_PER_ROW)]
                sq = vs[0] * vs[0]
                for k in range(1, VECS_PER_ROW):
                    sq = sq + vs[k] * vs[k]
                ssv = _hsum(sq)
                ssv = jnp.maximum(ssv, jnp.float32(1e-24))
                factor = _rsqrt_newton(ssv) * jnp.float32(SCALE)
                for k in range(VECS_PER_ROW):
                    rows_v[r, pl.ds(k * LANES, LANES)] = vs[k] * factor
                return _

            lax.fori_loop(0, TILE_ROWS, row_body, None)

            pltpu.sync_copy(
                rows_v,
                out_hbm.at[pl.ds(out_row0 + t * TILE_ROWS, TILE_ROWS)],
            )
            return _

        lax.fori_loop(0, NUM_TILES, tile_body, None)

    return emb_kernel


_EMB_KERNEL = _make_kernel()


@jax.jit
def kernel(x, raw_embedding):
    idx2d = x.reshape(TOTAL_ROWS // IDX_COLS, IDX_COLS).astype(jnp.int32)
    out = _EMB_KERNEL(idx2d, raw_embedding)
    return out.reshape(x.shape[0], x.shape[1], EMBED_DIM)


# speedup vs baseline: 1.4460x; 1.4460x over previous
import functools
import math

import jax
import jax.numpy as jnp
from jax import lax
from jax.experimental import pallas as pl
from jax.experimental.pallas import tpu as pltpu
from jax.experimental.pallas import tpu_sc as plsc

NUM_EMBEDDINGS = 1000000
EMBED_DIM = 64
TOTAL_ROWS = 4096 * 200
NUM_WORKERS = 32
ROWS_PER_WORKER = TOTAL_ROWS // NUM_WORKERS
TILE_ROWS = 512
NUM_TILES = ROWS_PER_WORKER // TILE_ROWS
GATHER_CHUNK = 128
GATHERS_PER_TILE = TILE_ROWS // GATHER_CHUNK
IDX_COLS = 128
SCALE = math.sqrt(EMBED_DIM)
LANES = 16
VECS_PER_ROW = EMBED_DIM // LANES
UNROLL = 8


def _perm(v, idx):
    return lax.gather(
        v,
        idx[:, None],
        dimension_numbers=lax.GatherDimensionNumbers(
            offset_dims=(), collapsed_slice_dims=(0,), start_index_map=(0,)),
        slice_sizes=(1,),
        mode=lax.GatherScatterMode.PROMISE_IN_BOUNDS,
    )


def _hsum(v):
    lanes = lax.iota(jnp.int32, LANES)
    for k in (8, 4, 2, 1):
        v = v + _perm(v, lanes ^ k)
    return v


def _rsqrt_newton(s):
    i = lax.bitcast_convert_type(s, jnp.int32)
    i = jnp.int32(0x5F3759DF) - lax.shift_right_arithmetic(i, 1)
    y = lax.bitcast_convert_type(i, jnp.float32)
    half = jnp.float32(0.5) * s
    for _ in range(2):
        y = y * (jnp.float32(1.5) - half * y * y)
    return y


def _make_kernel():
    mesh = plsc.VectorSubcoreMesh(core_axis_name="c", subcore_axis_name="s")

    @functools.partial(
        pl.kernel,
        mesh=mesh,
        compiler_params=pltpu.CompilerParams(use_tc_tiling_on_sc=False),
        out_type=jax.ShapeDtypeStruct((TOTAL_ROWS, EMBED_DIM), jnp.float32),
        scratch_types=[
            pltpu.VMEM((GATHERS_PER_TILE, GATHER_CHUNK), jnp.int32),
            pltpu.VMEM((TILE_ROWS, EMBED_DIM), jnp.float32),
            pltpu.SemaphoreType.DMA,
        ],
    )
    def emb_kernel(idx_hbm, table_hbm, out_hbm, idx_v, rows_v, sem):
        wid = lax.axis_index("s") * 2 + lax.axis_index("c")
        idx_row0 = wid * (ROWS_PER_WORKER // IDX_COLS)
        out_row0 = wid * ROWS_PER_WORKER

        def tile_body(t, _):
            pltpu.sync_copy(
                idx_hbm.at[pl.ds(idx_row0 + t * GATHERS_PER_TILE,
                                 GATHERS_PER_TILE)],
                idx_v,
            )
            copies = []
            for g in range(GATHERS_PER_TILE):
                copies.append(
                    pltpu.async_copy(
                        table_hbm.at[idx_v.at[g]],
                        rows_v.at[pl.ds(g * GATHER_CHUNK, GATHER_CHUNK)],
                        sem,
                    )
                )
            for c in copies:
                c.wait()

            def row_body(i, _):
                r0 = i * UNROLL
                for u in range(UNROLL):
                    r = r0 + u
                    vs = [rows_v[r, pl.ds(k * LANES, LANES)]
                          for k in range(VECS_PER_ROW)]
                    sq = vs[0] * vs[0]
                    for k in range(1, VECS_PER_ROW):
                        sq = sq + vs[k] * vs[k]
                    ssv = _hsum(sq)
                    ssv = jnp.maximum(ssv, jnp.float32(1e-24))
                    factor = _rsqrt_newton(ssv) * jnp.float32(SCALE)
                    for k in range(VECS_PER_ROW):
                        rows_v[r, pl.ds(k * LANES, LANES)] = vs[k] * factor
                return _

            lax.fori_loop(0, TILE_ROWS // UNROLL, row_body, None)

            pltpu.sync_copy(
                rows_v,
                out_hbm.at[pl.ds(out_row0 + t * TILE_ROWS, TILE_ROWS)],
            )
            return _

        lax.fori_loop(0, NUM_TILES, tile_body, None)

    return emb_kernel


_EMB_KERNEL = _make_kernel()


@jax.jit
def kernel(x, raw_embedding):
    idx2d = x.reshape(TOTAL_ROWS // IDX_COLS, IDX_COLS).astype(jnp.int32)
    out = _EMB_KERNEL(idx2d, raw_embedding)
    return out.reshape(x.shape[0], x.shape[1], EMBED_DIM)


# speedup vs baseline: 1.5849x; 1.0961x over previous
import functools
import math

import jax
import jax.numpy as jnp
from jax import lax
from jax.experimental import pallas as pl
from jax.experimental.pallas import tpu as pltpu
from jax.experimental.pallas import tpu_sc as plsc

NUM_EMBEDDINGS = 1000000
EMBED_DIM = 64
TOTAL_ROWS = 4096 * 200
NUM_WORKERS = 32
ROWS_PER_WORKER = TOTAL_ROWS // NUM_WORKERS
TILE_ROWS = 256
NUM_TILES = ROWS_PER_WORKER // TILE_ROWS
GATHER_CHUNK = 128
GATHERS_PER_TILE = TILE_ROWS // GATHER_CHUNK
IDX_COLS = 128
IDX_ROWS_PER_TILE = TILE_ROWS // IDX_COLS
NBUF = 4
SCALE = math.sqrt(EMBED_DIM)
LANES = 16
VECS_PER_ROW = EMBED_DIM // LANES
UNROLL = 8


def _perm(v, idx):
    return lax.gather(
        v,
        idx[:, None],
        dimension_numbers=lax.GatherDimensionNumbers(
            offset_dims=(), collapsed_slice_dims=(0,), start_index_map=(0,)),
        slice_sizes=(1,),
        mode=lax.GatherScatterMode.PROMISE_IN_BOUNDS,
    )


def _hsum(v):
    lanes = lax.iota(jnp.int32, LANES)
    for k in (8, 4, 2, 1):
        v = v + _perm(v, lanes ^ k)
    return v


def _rsqrt_newton(s):
    i = lax.bitcast_convert_type(s, jnp.int32)
    i = jnp.int32(0x5F3759DF) - lax.shift_right_arithmetic(i, 1)
    y = lax.bitcast_convert_type(i, jnp.float32)
    half = jnp.float32(0.5) * s
    for _ in range(2):
        y = y * (jnp.float32(1.5) - half * y * y)
    return y


def _make_kernel():
    mesh = plsc.VectorSubcoreMesh(core_axis_name="c", subcore_axis_name="s")

    @functools.partial(
        pl.kernel,
        mesh=mesh,
        compiler_params=pltpu.CompilerParams(use_tc_tiling_on_sc=False),
        out_type=jax.ShapeDtypeStruct((TOTAL_ROWS, EMBED_DIM), jnp.float32),
        scratch_types=[
            pltpu.VMEM((NBUF, GATHERS_PER_TILE, GATHER_CHUNK), jnp.int32),
            pltpu.VMEM((NBUF, TILE_ROWS, EMBED_DIM), jnp.float32),
        ]
        + [pltpu.SemaphoreType.DMA] * (2 * NBUF),
    )
    def emb_kernel(idx_hbm, table_hbm, out_hbm, idx_v, rows_v, *sems):
        gsems = sems[:NBUF]
        osems = sems[NBUF:]
        wid = lax.axis_index("s") * 2 + lax.axis_index("c")
        idx_row0 = wid * (ROWS_PER_WORKER // IDX_COLS)
        out_row0 = wid * ROWS_PER_WORKER

        def idx_copy(t, p):
            pltpu.sync_copy(
                idx_hbm.at[pl.ds(idx_row0 + t * IDX_ROWS_PER_TILE,
                                 IDX_ROWS_PER_TILE)],
                idx_v.at[p],
            )

        def fire_gathers(p):
            for g in range(GATHERS_PER_TILE):
                pltpu.async_copy(
                    table_hbm.at[idx_v.at[p, g]],
                    rows_v.at[p, pl.ds(g * GATHER_CHUNK, GATHER_CHUNK)],
                    gsems[p],
                )

        def wait_gathers(p):
            for g in range(GATHERS_PER_TILE):
                pltpu.make_async_copy(
                    table_hbm.at[idx_v.at[p, g]],
                    rows_v.at[p, pl.ds(g * GATHER_CHUNK, GATHER_CHUNK)],
                    gsems[p],
                ).wait()

        def fire_out(t, p):
            pltpu.async_copy(
                rows_v.at[p],
                out_hbm.at[pl.ds(out_row0 + t * TILE_ROWS, TILE_ROWS)],
                osems[p],
            )

        def wait_out(p):
            pltpu.make_async_copy(
                rows_v.at[p],
                out_hbm.at[pl.ds(out_row0, TILE_ROWS)],
                osems[p],
            ).wait()

        def compute(p):
            def row_body(i, _):
                r0 = i * UNROLL
                for u in range(UNROLL):
                    r = r0 + u
                    vs = [rows_v[p, r, pl.ds(k * LANES, LANES)]
                          for k in range(VECS_PER_ROW)]
                    sq = vs[0] * vs[0]
                    for k in range(1, VECS_PER_ROW):
                        sq = sq + vs[k] * vs[k]
                    ssv = _hsum(sq)
                    ssv = jnp.maximum(ssv, jnp.float32(1e-24))
                    factor = _rsqrt_newton(ssv) * jnp.float32(SCALE)
                    for k in range(VECS_PER_ROW):
                        rows_v[p, r, pl.ds(k * LANES, LANES)] = vs[k] * factor
                return _

            lax.fori_loop(0, TILE_ROWS // UNROLL, row_body, None)

        for t0 in range(2):
            idx_copy(jnp.int32(t0), t0)
            fire_gathers(t0)

        def quad_body(i, _):
            for par in range(NBUF):
                t = i * NBUF + par
                wait_gathers(par)
                compute(par)
                fire_out(t, par)
                q = (par + 2) % NBUF

                @pl.when(t >= 2)
                def _():
                    wait_out(q)

                @pl.when(t + 2 < NUM_TILES)
                def _():
                    idx_copy(t + 2, q)
                    fire_gathers(q)
            return _

        lax.fori_loop(0, NUM_TILES // NBUF, quad_body, None)

        wait_out((NUM_TILES - 2) % NBUF)
        wait_out((NUM_TILES - 1) % NBUF)

    return emb_kernel


_EMB_KERNEL = _make_kernel()


@jax.jit
def kernel(x, raw_embedding):
    idx2d = x.reshape(TOTAL_ROWS // IDX_COLS, IDX_COLS).astype(jnp.int32)
    out = _EMB_KERNEL(idx2d, raw_embedding)
    return out.reshape(x.shape[0], x.shape[1], EMBED_DIM)


# speedup vs baseline: 1.6978x; 1.0712x over previous
import functools
import math

import jax
import jax.numpy as jnp
from jax import lax
from jax.experimental import pallas as pl
from jax.experimental.pallas import tpu as pltpu
from jax.experimental.pallas import tpu_sc as plsc

NUM_EMBEDDINGS = 1000000
EMBED_DIM = 64
BATCH = 4096
SEQ = 200
NUM_WORKERS = 32
B_PER_WORKER = BATCH // NUM_WORKERS
TILE_B = 2
TILE_ROWS = TILE_B * SEQ
NUM_TILES = B_PER_WORKER // TILE_B
CHUNKS = ((0, 104), (104, 96))
NBUF = 4
SCALE = math.sqrt(EMBED_DIM)
LANES = 16
VECS_PER_ROW = EMBED_DIM // LANES
UNROLL = 8


def _perm(v, idx):
    return lax.gather(
        v,
        idx[:, None],
        dimension_numbers=lax.GatherDimensionNumbers(
            offset_dims=(), collapsed_slice_dims=(0,), start_index_map=(0,)),
        slice_sizes=(1,),
        mode=lax.GatherScatterMode.PROMISE_IN_BOUNDS,
    )


def _rsqrt_newton(s):
    i = lax.bitcast_convert_type(s, jnp.int32)
    i = jnp.int32(0x5F3759DF) - lax.shift_right_arithmetic(i, 1)
    y = lax.bitcast_convert_type(i, jnp.float32)
    half = jnp.float32(0.5) * s
    for _ in range(2):
        y = y * (jnp.float32(1.5) - half * y * y)
    return y


def _make_kernel():
    mesh = plsc.VectorSubcoreMesh(core_axis_name="c", subcore_axis_name="s")

    @functools.partial(
        pl.kernel,
        mesh=mesh,
        compiler_params=pltpu.CompilerParams(use_tc_tiling_on_sc=False),
        out_type=jax.ShapeDtypeStruct((BATCH, SEQ, EMBED_DIM), jnp.float32),
        scratch_types=[
            pltpu.VMEM((NBUF, TILE_B, SEQ), jnp.int32),
            pltpu.VMEM((NBUF, TILE_B, SEQ, EMBED_DIM), jnp.float32),
        ]
        + [pltpu.SemaphoreType.DMA] * (2 * NBUF),
    )
    def emb_kernel(idx_hbm, table_hbm, out_hbm, idx_v, rows_v, *sems):
        gsems = sems[:NBUF]
        osems = sems[NBUF:]
        wid = lax.axis_index("s") * 2 + lax.axis_index("c")
        b0 = wid * B_PER_WORKER

        def idx_copy(t, p):
            pltpu.sync_copy(
                idx_hbm.at[pl.ds(b0 + t * TILE_B, TILE_B)],
                idx_v.at[p],
            )

        def gather_parts(p):
            for j in range(TILE_B):
                for (off, n) in CHUNKS:
                    yield (
                        table_hbm.at[idx_v.at[p, j, pl.ds(off, n)]],
                        rows_v.at[p, j, pl.ds(off, n)],
                        gsems[p],
                    )

        def fire_gathers(p):
            for src, dst, sem in gather_parts(p):
                pltpu.async_copy(src, dst, sem)

        def wait_gathers(p):
            for src, dst, sem in gather_parts(p):
                pltpu.make_async_copy(src, dst, sem).wait()

        def fire_out(t, p):
            pltpu.async_copy(
                rows_v.at[p],
                out_hbm.at[pl.ds(b0 + t * TILE_B, TILE_B)],
                osems[p],
            )

        def wait_out(p):
            pltpu.make_async_copy(
                rows_v.at[p],
                out_hbm.at[pl.ds(b0, TILE_B)],
                osems[p],
            ).wait()

        def compute(p):
            lanes = lax.iota(jnp.int32, LANES)

            def row_pack(j, r0):
                vs = [[rows_v[p, j, r0 + q, pl.ds(k * LANES, LANES)]
                       for k in range(VECS_PER_ROW)] for q in range(4)]
                f = []
                for q in range(4):
                    sq = vs[q][0] * vs[q][0]
                    for k in range(1, VECS_PER_ROW):
                        sq = sq + vs[q][k] * vs[q][k]
                    g = sq + _perm(sq, lanes ^ 8)
                    g = g + _perm(g, lanes ^ 4)
                    f.append(g)
                m = jnp.where(lanes < 4, f[0],
                              jnp.where(lanes < 8, f[1],
                                        jnp.where(lanes < 12, f[2], f[3])))
                m = m + _perm(m, lanes ^ 2)
                m = m + _perm(m, lanes ^ 1)
                m = jnp.maximum(m, jnp.float32(1e-24))
                y = _rsqrt_newton(m) * jnp.float32(SCALE)
                for q in range(4):
                    fac = _perm(y, jnp.full((LANES,), 4 * q, jnp.int32))
                    for k in range(VECS_PER_ROW):
                        rows_v[p, j, r0 + q, pl.ds(k * LANES, LANES)] = (
                            vs[q][k] * fac)

            def row_body(i, _):
                for u in range(UNROLL // 4):
                    row_pack(0, i * UNROLL + u * 4)
                    row_pack(1, i * UNROLL + u * 4)
                return _

            lax.fori_loop(0, SEQ // UNROLL, row_body, None)

        for t0 in range(2):
            idx_copy(jnp.int32(t0), t0)
            fire_gathers(t0)

        def quad_body(i, _):
            for par in range(NBUF):
                t = i * NBUF + par
                wait_gathers(par)
                compute(par)
                fire_out(t, par)
                q = (par + 2) % NBUF

                @pl.when(t >= 2)
                def _():
                    wait_out(q)

                @pl.when(t + 2 < NUM_TILES)
                def _():
                    idx_copy(t + 2, q)
                    fire_gathers(q)
            return _

        lax.fori_loop(0, NUM_TILES // NBUF, quad_body, None)

        wait_out((NUM_TILES - 2) % NBUF)
        wait_out((NUM_TILES - 1) % NBUF)

    return emb_kernel


_EMB_KERNEL = _make_kernel()


@jax.jit
def kernel(x, raw_embedding):
    return _EMB_KERNEL(x.astype(jnp.int32), raw_embedding)
